# baseline (speedup 1.0000x reference)
import functools

import jax
import jax.numpy as jnp
from jax import lax
from jax.experimental import pallas as pl
from jax.experimental.pallas import tpu as pltpu
from jax.experimental.pallas import tpu_sc as plsc

NC = 2
NS = 16
L = 16
NW = NC * NS

B = 16384
D = 64
DP = 128
V = 1010
BPW = B // NW
CH = 128
NCH = BPW // CH
GPC = CH // L

_MAGIC = 0x5F3759DF
_EPS2 = 1e-24


def _rsqrt(x):
    i = plsc.bitcast(x, jnp.int32)
    y = plsc.bitcast(_MAGIC - (i >> 1), jnp.float32)
    for _ in range(3):
        y = y * (1.5 - 0.5 * x * y * y)
    return y


_mesh = plsc.VectorSubcoreMesh(
    core_axis_name="c", subcore_axis_name="s", num_cores=NC, num_subcores=NS
)


@functools.partial(
    pl.kernel,
    out_type=jax.ShapeDtypeStruct((B,), jnp.float32),
    mesh=_mesh,
    compiler_params=pltpu.CompilerParams(
        needs_layout_passes=False,
        use_tc_tiling_on_sc=True,
        disable_bounds_checks=True,
    ),
    scratch_types=[
        pltpu.VMEM((BPW,), jnp.int32),
        pltpu.VMEM((CH, DP), jnp.float32),
        pltpu.VMEM((CH, DP), jnp.float32),
        pltpu.VMEM((CH // 2, DP), jnp.float32),
        pltpu.VMEM((CH // 2, DP), jnp.float32),
        pltpu.VMEM((BPW,), jnp.float32),
        pltpu.SemaphoreType.DMA,
        pltpu.SemaphoreType.DMA,
        pltpu.SemaphoreType.DMA,
        pltpu.SemaphoreType.DMA,
    ],
)
def _cosdist(
    rep_hbm, expr_hbm, table_hbm, out_hbm,
    idx_v, rows_b0, rows_b1, rep_b0, rep_b1, out_v,
    sem_g0, sem_g1, sem_r0, sem_r1,
):
    wid = lax.axis_index("s") * NC + lax.axis_index("c")
    base = wid * BPW

    rows_b = (rows_b0, rows_b1)
    rep_b = (rep_b0, rep_b1)
    sem_g = (sem_g0, sem_g1)
    sem_r = (sem_r0, sem_r1)

    pltpu.sync_copy(expr_hbm.at[pl.ds(base, BPW)], idx_v)

    def start(c):
        s = c % 2
        g = pltpu.async_copy(
            table_hbm.at[idx_v.at[pl.ds(c * CH, CH)]], rows_b[s], sem_g[s]
        )
        r = pltpu.async_copy(
            rep_hbm.at[pl.ds(pl.multiple_of((base + c * CH) // 2, 8), CH // 2)],
            rep_b[s],
            sem_r[s],
        )
        return g, r

    lanes = lax.iota(jnp.int32, L)
    pending = start(0)

    for c in range(NCH):
        s = c % 2
        nxt = start(c + 1) if c + 1 < NCH else None
        pending[0].wait()
        pending[1].wait()
        rows_v = rows_b[s]
        rep_v = rep_b[s]

        @plsc.parallel_loop(0, GPC, 1, unroll=2)
        def group(g, rows_v=rows_v, rep_v=rep_v, c=c):
            row0 = g * L
            zero = jnp.zeros((L,), jnp.float32)
            xy, xx, yy = zero, zero, zero
            for i in range(L):
                r = row0 + i
                pxy, pxx, pyy = zero, zero, zero
                for k in range(D // L):
                    xv = rows_v[r, pl.ds(k * L, L)]
                    yv = rep_v[row0 // 2 + i // 2, pl.ds((i % 2) * D + k * L, L)]
                    pxy += xv * yv
                    pxx += xv * xv
                    pyy += yv * yv
                here = lanes == i
                xy = jnp.where(here, jnp.sum(pxy), xy)
                xx = jnp.where(here, jnp.sum(pxx), xx)
                yy = jnp.where(here, jnp.sum(pyy), yy)
            rr = _rsqrt(jnp.maximum(xx, _EPS2)) * _rsqrt(jnp.maximum(yy, _EPS2))
            out_v[pl.ds(c * CH + row0, L)] = 1.0 - xy * rr
        pending = nxt

    pltpu.sync_copy(out_v, out_hbm.at[pl.ds(base, BPW)])


def kernel(rep, expr, emb_weight):
    tpad = jnp.pad(emb_weight, ((0, 0), (0, DP - D)))
    return _cosdist(rep.reshape(B // 2, 2 * D), expr, tpad)


# speedup vs baseline: 1.1193x; 1.1193x over previous
import functools

import jax
import jax.numpy as jnp
from jax import lax
from jax.experimental import pallas as pl
from jax.experimental.pallas import tpu as pltpu
from jax.experimental.pallas import tpu_sc as plsc

NC = 2
NS = 16
L = 16
NW = NC * NS

B = 16384
D = 64
DP = 128
V = 1010
BPW = B // NW
CH = 128
NCH = BPW // CH
GPC = CH // L

_MAGIC = 0x5F3759DF
_EPS2 = 1e-24


def _rsqrt(x):
    i = plsc.bitcast(x, jnp.int32)
    y = plsc.bitcast(_MAGIC - (i >> 1), jnp.float32)
    for _ in range(3):
        y = y * (1.5 - 0.5 * x * y * y)
    return y


_mesh = plsc.VectorSubcoreMesh(
    core_axis_name="c", subcore_axis_name="s", num_cores=NC, num_subcores=NS
)


@functools.partial(
    pl.kernel,
    out_type=jax.ShapeDtypeStruct((B,), jnp.float32),
    mesh=_mesh,
    compiler_params=pltpu.CompilerParams(
        needs_layout_passes=False,
        use_tc_tiling_on_sc=True,
        disable_bounds_checks=True,
    ),
    scratch_types=[
        pltpu.VMEM((BPW,), jnp.int32),
        pltpu.VMEM((CH, DP), jnp.float32),
        pltpu.VMEM((CH, DP), jnp.float32),
        pltpu.VMEM((CH, D), jnp.float32),
        pltpu.VMEM((CH, D), jnp.float32),
        pltpu.VMEM((BPW,), jnp.float32),
        pltpu.SemaphoreType.DMA,
        pltpu.SemaphoreType.DMA,
        pltpu.SemaphoreType.DMA,
        pltpu.SemaphoreType.DMA,
    ],
)
def _cosdist(
    rep_hbm, expr_hbm, table_hbm, out_hbm,
    idx_v, rows_b0, rows_b1, rep_b0, rep_b1, out_v,
    sem_g0, sem_g1, sem_r0, sem_r1,
):
    wid = lax.axis_index("s") * NC + lax.axis_index("c")
    base = wid * BPW

    rows_b = (rows_b0, rows_b1)
    rep_b = (rep_b0, rep_b1)
    sem_g = (sem_g0, sem_g1)
    sem_r = (sem_r0, sem_r1)

    pltpu.sync_copy(expr_hbm.at[pl.ds(base, BPW)], idx_v)

    def start(c):
        s = c % 2
        g = pltpu.async_copy(
            table_hbm.at[idx_v.at[pl.ds(c * CH, CH)]], rows_b[s], sem_g[s]
        )
        r = pltpu.async_copy(
            rep_hbm.at[pl.ds(base + c * CH, CH)], rep_b[s], sem_r[s]
        )
        return g, r

    lanes = lax.iota(jnp.int32, L)
    pending = start(0)

    for c in range(NCH):
        s = c % 2
        nxt = start(c + 1) if c + 1 < NCH else None
        pending[0].wait()
        pending[1].wait()
        rows_v = rows_b[s]
        rep_v = rep_b[s]

        @plsc.parallel_loop(0, GPC, 1, unroll=2)
        def group(g, rows_v=rows_v, rep_v=rep_v, c=c):
            row0 = g * L
            zero = jnp.zeros((L,), jnp.float32)
            xy, xx, yy = zero, zero, zero
            for i in range(L):
                r = row0 + i
                pxy, pxx, pyy = zero, zero, zero
                for k in range(D // L):
                    xv = rows_v[r, pl.ds(k * L, L)]
                    yv = rep_v[r, pl.ds(k * L, L)]
                    pxy += xv * yv
                    pxx += xv * xv
                    pyy += yv * yv
                here = lanes == i
                xy = jnp.where(here, jnp.sum(pxy), xy)
                xx = jnp.where(here, jnp.sum(pxx), xx)
                yy = jnp.where(here, jnp.sum(pyy), yy)
            rr = _rsqrt(jnp.maximum(xx, _EPS2)) * _rsqrt(jnp.maximum(yy, _EPS2))
            out_v[pl.ds(c * CH + row0, L)] = 1.0 - xy * rr
        pending = nxt

    pltpu.sync_copy(out_v, out_hbm.at[pl.ds(base, BPW)])


def kernel(rep, expr, emb_weight):
    tpad = jnp.pad(emb_weight, ((0, 0), (0, DP - D)))
    return _cosdist(rep, expr, tpad)
